# Initial kernel scaffold; baseline (speedup 1.0000x reference)
import functools

import jax
import jax.numpy as jnp
from jax import lax
from jax.experimental import pallas as pl
from jax.experimental.pallas import tpu as pltpu
from jax.experimental.pallas import tpu_sc as plsc

N, E, D = 10000, 320000, 128

NC, NS = 2, 16
NW = NC * NS
EPW = E // NW
CH = 80
NCHUNK = EPW // CH
ROWS_PER_TILE = N // NS


def _agg_body(feats, src, dst, zeros, out, src_v, dst_v, rows_v, acc_sh, sem):
    c = lax.axis_index("c")
    s = lax.axis_index("s")
    wid = c * NS + s

    r0 = pl.multiple_of(s * ROWS_PER_TILE, 8)
    pltpu.sync_copy(zeros.at[pl.ds(r0, ROWS_PER_TILE)],
                    acc_sh.at[pl.ds(r0, ROWS_PER_TILE)])
    plsc.subcore_barrier()

    e_base = wid * EPW

    def step(i, _):
        base = pl.multiple_of(e_base + i * CH, 8)
        pltpu.sync_copy(src.at[pl.ds(base, CH)], src_v)
        pltpu.sync_copy(dst.at[pl.ds(base, CH)], dst_v)
        pltpu.async_copy(feats.at[src_v], rows_v, sem).wait()
        pltpu.sync_copy(rows_v, acc_sh.at[dst_v], add=True)
        return ()

    lax.fori_loop(0, NCHUNK, step, (), unroll=False)

    plsc.subcore_barrier()
    pltpu.sync_copy(acc_sh.at[pl.ds(r0, ROWS_PER_TILE)],
                    out.at[c, pl.ds(r0, ROWS_PER_TILE)])


_agg_sc = functools.partial(
    pl.kernel,
    out_type=jax.ShapeDtypeStruct((NC, N, D), jnp.float32),
    mesh=plsc.VectorSubcoreMesh(core_axis_name="c", subcore_axis_name="s",
                                num_cores=NC, num_subcores=NS),
    scratch_types=[
        pltpu.VMEM((CH,), jnp.int32),
        pltpu.VMEM((CH,), jnp.int32),
        pltpu.VMEM((CH, D), jnp.float32),
        pltpu.VMEM_SHARED((N, D), jnp.float32),
        pltpu.SemaphoreType.DMA,
    ],
)(_agg_body)


BI = 400
GRID = N // BI


def _tc_body(x_ref, agg_ref, eps_ref, w1_ref, b1_ref, w2_ref, b2_ref,
             out_ref, h_ref):
    i = pl.program_id(0)

    @pl.when(i == 0)
    def _():
        agg = agg_ref[0] + agg_ref[1]
        h0 = (1.0 + eps_ref[0, 0]) * x_ref[...] + agg
        h1 = jnp.maximum(
            jnp.dot(h0, w1_ref[...], preferred_element_type=jnp.float32)
            + b1_ref[...], 0.0)
        h2 = jnp.maximum(
            jnp.dot(h1, w2_ref[...], preferred_element_type=jnp.float32)
            + b2_ref[...], 0.0)
        h_ref[...] = h2

    hb = h_ref[pl.ds(i * BI, BI), :]
    out_ref[...] = lax.dot_general(hb, h_ref[...], (((1,), (1,)), ((), ())),
                                   preferred_element_type=jnp.float32)


def _tc_call(x, agg2, eps11, W1, b1r, W2, b2r):
    full = lambda shape: pl.BlockSpec(shape, lambda i, _s=None: (0,) * len(shape))
    return pl.pallas_call(
        _tc_body,
        grid=(GRID,),
        in_specs=[
            full((N, D)),
            full((NC, N, D)),
            full((1, 1)),
            full((D, D)),
            full((1, D)),
            full((D, D)),
            full((1, D)),
        ],
        out_specs=pl.BlockSpec((BI, N), lambda i: (i, 0)),
        out_shape=jax.ShapeDtypeStruct((N, N), jnp.float32),
        scratch_shapes=[pltpu.VMEM((N, D), jnp.float32)],
    )(x, agg2, eps11, W1, b1r, W2, b2r)


def kernel(node_feats, edge_idx, eps, W1, b1, W2, b2):
    src = edge_idx[0]
    dst = edge_idx[1]
    zeros = jnp.zeros((N, D), jnp.float32)
    agg2 = _agg_sc(node_feats, src, dst, zeros)
    return _tc_call(node_feats, agg2, eps.reshape(1, 1), W1,
                    b1.reshape(1, D), W2, b2.reshape(1, D))


# speedup vs baseline: 3.9776x; 3.9776x over previous
import functools

import jax
import jax.numpy as jnp
from jax import lax
from jax.experimental import pallas as pl
from jax.experimental.pallas import tpu as pltpu
from jax.experimental.pallas import tpu_sc as plsc

N, E, D = 10000, 320000, 128

NC, NS = 2, 16
NW = NC * NS
EPW = E // NW
CH = 80
NCHUNK = EPW // CH
N_PAD = 10240
ROWS_PER_TILE = N_PAD // NS


def _agg_body(feats, src, dst, zeros, out, src_v, dst_v, rows_v, acc_sh, sem):
    c = lax.axis_index("c")
    s = lax.axis_index("s")
    wid = c * NS + s

    r0 = pl.multiple_of(s * ROWS_PER_TILE, 8)
    pltpu.sync_copy(zeros.at[pl.ds(r0, ROWS_PER_TILE)],
                    acc_sh.at[pl.ds(r0, ROWS_PER_TILE)])
    plsc.subcore_barrier()

    e_base = wid * EPW

    def step(i, _):
        base = pl.multiple_of(e_base + i * CH, 8)
        pltpu.sync_copy(src.at[pl.ds(base, CH)], src_v)
        pltpu.sync_copy(dst.at[pl.ds(base, CH)], dst_v)
        pltpu.async_copy(feats.at[src_v], rows_v, sem).wait()
        pltpu.sync_copy(rows_v, acc_sh.at[dst_v], add=True)
        return ()

    lax.fori_loop(0, NCHUNK, step, (), unroll=False)

    plsc.subcore_barrier()
    pltpu.sync_copy(acc_sh.at[pl.ds(r0, ROWS_PER_TILE)],
                    out.at[c, pl.ds(r0, ROWS_PER_TILE)])


@functools.cache
def _agg_sc():
    return pl.kernel(
        _agg_body,
        out_type=jax.ShapeDtypeStruct((NC, N_PAD, D), jnp.float32),
        mesh=plsc.VectorSubcoreMesh(core_axis_name="c", subcore_axis_name="s",
                                    num_cores=NC, num_subcores=NS),
        scratch_types=[
            pltpu.VMEM((CH,), jnp.int32),
            pltpu.VMEM((CH,), jnp.int32),
            pltpu.VMEM((CH, D), jnp.float32),
            pltpu.VMEM_SHARED((N_PAD, D), jnp.float32),
            pltpu.SemaphoreType.DMA,
        ],
    )


BI = 400
GRID = N // BI


def _tc_body(x_ref, agg_ref, eps_ref, w1_ref, b1_ref, w2_ref, b2_ref,
             out_ref, h_ref):
    i = pl.program_id(0)

    @pl.when(i == 0)
    def _():
        agg = agg_ref[0, :N, :] + agg_ref[1, :N, :]
        h0 = (1.0 + eps_ref[0, 0]) * x_ref[...] + agg
        h1 = jnp.maximum(
            jnp.dot(h0, w1_ref[...], preferred_element_type=jnp.float32)
            + b1_ref[...], 0.0)
        h2 = jnp.maximum(
            jnp.dot(h1, w2_ref[...], preferred_element_type=jnp.float32)
            + b2_ref[...], 0.0)
        h_ref[...] = h2

    hb = h_ref[pl.ds(i * BI, BI), :]
    out_ref[...] = lax.dot_general(hb, h_ref[...], (((1,), (1,)), ((), ())),
                                   preferred_element_type=jnp.float32)


def _tc_call(x, agg2, eps11, W1, b1r, W2, b2r):
    full = lambda shape: pl.BlockSpec(shape, lambda i, _s=None: (0,) * len(shape))
    return pl.pallas_call(
        _tc_body,
        grid=(GRID,),
        in_specs=[
            full((N, D)),
            full((NC, N_PAD, D)),
            full((1, 1)),
            full((D, D)),
            full((1, D)),
            full((D, D)),
            full((1, D)),
        ],
        out_specs=pl.BlockSpec((BI, N), lambda i: (i, 0)),
        out_shape=jax.ShapeDtypeStruct((N, N), jnp.float32),
        scratch_shapes=[pltpu.VMEM((N, D), jnp.float32)],
    )(x, agg2, eps11, W1, b1r, W2, b2r)


def kernel(node_feats, edge_idx, eps, W1, b1, W2, b2):
    src = edge_idx[0]
    dst = edge_idx[1]
    zeros = jnp.zeros((N_PAD, D), jnp.float32)
    agg2 = _agg_sc()(node_feats, src, dst, zeros)
    return _tc_call(node_feats, agg2, eps.reshape(1, 1), W1,
                    b1.reshape(1, D), W2, b2.reshape(1, D))
